# baseline (speedup 1.0000x reference)
import jax
import jax.numpy as jnp
from jax.experimental import pallas as pl

_SZ = 2048
_TEMP = 10.0
_EPS = 1e-20
_BR = 256


def _body(a0_ref, a1_ref, u0_ref, u1_ref, o_ref):
    l0 = _EPS - jnp.log(u0_ref[...] + _EPS)
    l1 = _EPS - jnp.log(u1_ref[...] + _EPS)
    x = (a0_ref[...] - a1_ref[...] + jnp.log(l1 / l0)) * (1.0 / _TEMP)
    o_ref[...] = jax.nn.sigmoid(x)


def kernel(gen_matrix, u):
    a0 = gen_matrix[:, :, 0]
    a1 = gen_matrix[:, :, 1]
    u0 = u[:, 0].reshape(_SZ, _SZ)
    u1 = u[:, 1].reshape(_SZ, _SZ)
    spec = pl.BlockSpec((_BR, _SZ), lambda i: (i, 0))
    return pl.pallas_call(
        _body,
        grid=(_SZ // _BR,),
        in_specs=[spec, spec, spec, spec],
        out_specs=spec,
        out_shape=jax.ShapeDtypeStruct((_SZ, _SZ), jnp.float32),
    )(a0, a1, u0, u1)
